# baseline (speedup 1.0000x reference)
import functools

import jax
import jax.numpy as jnp
from jax import lax
from jax.experimental import pallas as pl
from jax.experimental.pallas import tpu as pltpu
from jax.experimental.pallas import tpu_sc as plsc

_NUM_CORES = 2
_NUM_SUBCORES = 16
_NW = _NUM_CORES * _NUM_SUBCORES
_CHUNK = 64
_L = 16


@functools.partial(jax.jit, static_argnums=(2, 3))
def _gather_sc(weight3, x, b_per_w, d):
    n_chunks = b_per_w // _CHUNK
    assert n_chunks % 2 == 0
    mesh = plsc.VectorSubcoreMesh(core_axis_name="c", subcore_axis_name="s")

    @functools.partial(
        pl.kernel,
        mesh=mesh,
        out_type=jax.ShapeDtypeStruct((_NW * b_per_w, d), jnp.float32),
        scratch_types=[
            pltpu.VMEM((b_per_w,), jnp.int32),
            pltpu.VMEM((_CHUNK, d), jnp.float32),
            pltpu.VMEM((_CHUNK, d), jnp.float32),
            pltpu.SemaphoreType.DMA,
            pltpu.SemaphoreType.DMA,
        ],
    )
    def k(table_hbm, idx_hbm, out_hbm, idx_v, buf_a, buf_b, sem_a, sem_b):
        wid = lax.axis_index("s") * _NUM_CORES + lax.axis_index("c")
        base = wid * b_per_w

        def fire(c, buf, sem):
            for g in range(_CHUNK // _L):
                xv = idx_v[pl.ds(c * _CHUNK + g * _L, _L)]
                for l in range(_L):
                    x_sc = xv[l]
                    q = lax.shift_right_logical(x_sc, 3)
                    r = lax.rem(x_sc, 8)
                    pltpu.async_copy(
                        table_hbm.at[q, r], buf.at[g * _L + l], sem
                    )

        def drain(buf, sem):
            pltpu.make_async_copy(
                out_hbm.at[pl.ds(0, _CHUNK)], buf, sem
            ).wait()

        def write(c, buf):
            pltpu.sync_copy(buf, out_hbm.at[pl.ds(base + c * _CHUNK, _CHUNK)])

        pltpu.sync_copy(idx_hbm.at[pl.ds(base, b_per_w)], idx_v)
        fire(0, buf_a, sem_a)

        def pair_body(i, carry):
            fire(2 * i + 1, buf_b, sem_b)
            drain(buf_a, sem_a)
            write(2 * i, buf_a)

            @pl.when(i < (n_chunks // 2 - 1))
            def _():
                fire(2 * i + 2, buf_a, sem_a)

            drain(buf_b, sem_b)
            write(2 * i + 1, buf_b)
            return carry

        lax.fori_loop(0, n_chunks // 2, pair_body, 0)

    return k(weight3, x)


def kernel(x, weight):
    b = x.shape[0]
    d = weight.shape[1]
    b_per_w = b // _NW
    weight3 = weight.reshape(-1, 8, d)
    return _gather_sc(weight3, x, b_per_w, d)


# speedup vs baseline: 1.0056x; 1.0056x over previous
import functools

import jax
import jax.numpy as jnp
from jax import lax
from jax.experimental import pallas as pl
from jax.experimental.pallas import tpu as pltpu
from jax.experimental.pallas import tpu_sc as plsc

_NUM_CORES = 2
_NUM_SUBCORES = 16
_NW = _NUM_CORES * _NUM_SUBCORES
_CHUNK = 64
_L = 16


@functools.partial(jax.jit, static_argnums=(2, 3))
def _gather_sc(weight3, x, b_per_w, d):
    n_chunks = b_per_w // _CHUNK
    mesh = plsc.VectorSubcoreMesh(core_axis_name="c", subcore_axis_name="s")

    @functools.partial(
        pl.kernel,
        mesh=mesh,
        out_type=jax.ShapeDtypeStruct((_NW * b_per_w, d), jnp.float32),
        scratch_types=[
            pltpu.VMEM((b_per_w,), jnp.int32),
            pltpu.VMEM((_CHUNK, d), jnp.float32),
            pltpu.SemaphoreType.DMA,
        ],
    )
    def k(table_hbm, idx_hbm, out_hbm, idx_v, rowchunk_v, sem):
        wid = lax.axis_index("s") * _NUM_CORES + lax.axis_index("c")
        base = wid * b_per_w
        pltpu.sync_copy(idx_hbm.at[pl.ds(base, b_per_w)], idx_v)

        def chunk_body(j, carry):
            copies = []
            for g in range(_CHUNK // _L):
                xv = idx_v[pl.ds(j * _CHUNK + g * _L, _L)]
                for l in range(_L):
                    x_sc = xv[l]
                    q = lax.shift_right_logical(x_sc, 3)
                    r = lax.rem(x_sc, 8)
                    copies.append(
                        pltpu.async_copy(
                            table_hbm.at[q, r],
                            rowchunk_v.at[g * _L + l],
                            sem,
                        )
                    )
            for c in copies:
                c.wait()
            pltpu.sync_copy(
                rowchunk_v, out_hbm.at[pl.ds(base + j * _CHUNK, _CHUNK)]
            )
            return carry

        lax.fori_loop(0, n_chunks, chunk_body, 0)

    return k(weight3, x)


def kernel(x, weight):
    b = x.shape[0]
    d = weight.shape[1]
    b_per_w = b // _NW
    weight3 = weight.reshape(-1, 8, d)
    return _gather_sc(weight3, x, b_per_w, d)
